# baseline (speedup 1.0000x reference)
import functools

import jax
import jax.numpy as jnp
from jax.experimental import pallas as pl
from jax.experimental.pallas import tpu as pltpu


def _stg_kernel(x_ref, w_ref, b_ref, out_ref, scores_ref,
                selk_ref, selx_ref, mag_ref, *, bk: int, topk: int):
    m = pl.program_id(0)
    k = pl.program_id(1)
    nk = pl.num_programs(1)
    nm = pl.num_programs(0)
    topk_f = jnp.float32(topk)

    @pl.when(m > 0)
    def _select():
        @pl.when(k == 0)
        def _zero():
            mag_ref[...] = jnp.zeros_like(mag_ref)

        keys = selk_ref[...]

        def bit_step(i, mag):
            cand = mag + (jnp.int32(1) << (29 - i))
            cnt = jnp.sum((keys >= cand).astype(jnp.float32), axis=1,
                          keepdims=True)
            return jnp.where(cnt >= topk_f, cand, mag)

        @pl.when(k < nk - 1)
        def _bits():
            mag_ref[...] = jax.lax.fori_loop(
                4 * k, jnp.minimum(4 * k + 4, 30), bit_step, mag_ref[...])

        @pl.when(k == nk - 1)
        def _finish():
            start = 4 * (nk - 1)
            mag = jax.lax.fori_loop(
                jnp.minimum(start, 30), 30, bit_step, mag_ref[...])
            out_ref[...] = jnp.where(keys >= mag, selx_ref[...], 0.0)

    @pl.when(m < nm - 1)
    def _matmul():
        x_blk = x_ref[:, pl.ds(k * bk, bk)].astype(jnp.bfloat16)
        prod = jax.lax.dot_general(
            x_blk, w_ref[...],
            dimension_numbers=(((1,), (1,)), ((), ())),
            preferred_element_type=jnp.float32)

        @pl.when(k == 0)
        def _init():
            scores_ref[...] = prod

        @pl.when(k > 0)
        def _accum():
            scores_ref[...] += prod

        @pl.when(k == nk - 1)
        def _stage():
            scores = jax.nn.sigmoid(scores_ref[...] + b_ref[...])
            scores_ref[...] = scores
            selk_ref[...] = jax.lax.bitcast_convert_type(scores, jnp.int32)
            selx_ref[...] = x_ref[...]


def kernel(x, W, b):
    m, kdim = x.shape
    n = W.shape[0]
    topk = max(1, int(0.3 * n))
    bm = min(256, m)
    bk = min(512, kdim)
    nm = m // bm
    grid = (nm + 1, kdim // bk)
    last = nm - 1

    masked, scores = pl.pallas_call(
        functools.partial(_stg_kernel, bk=bk, topk=topk),
        grid=grid,
        in_specs=[
            pl.BlockSpec((bm, kdim), lambda i, k: (jnp.minimum(i, last), 0)),
            pl.BlockSpec((n, bk), lambda i, k: (0, k)),
            pl.BlockSpec((1, n), lambda i, k: (0, 0)),
        ],
        out_specs=[
            pl.BlockSpec((bm, n), lambda i, k: (jnp.maximum(i - 1, 0), 0)),
            pl.BlockSpec((bm, n), lambda i, k: (jnp.minimum(i, last), 0)),
        ],
        out_shape=[
            jax.ShapeDtypeStruct((m, n), jnp.float32),
            jax.ShapeDtypeStruct((m, n), jnp.float32),
        ],
        scratch_shapes=[
            pltpu.VMEM((bm, n), jnp.int32),
            pltpu.VMEM((bm, kdim), jnp.float32),
            pltpu.VMEM((bm, 1), jnp.int32),
        ],
        compiler_params=pltpu.CompilerParams(
            dimension_semantics=("arbitrary", "arbitrary")),
    )(x, W.astype(jnp.bfloat16), b.reshape(1, n))
    return (masked, scores)
